# baseline (speedup 1.0000x reference)
import functools

import jax
import jax.numpy as jnp
import numpy as np
from jax.experimental import pallas as pl
from jax.experimental.pallas import tpu as pltpu

SPARSITY = 0.02


def _top4_of_8(s):
    mx, mn = jnp.maximum, jnp.minimum
    h = [mx(s[2 * i], s[2 * i + 1]) for i in range(4)]
    l = [mn(s[2 * i], s[2 * i + 1]) for i in range(4)]
    quads = []
    for i in (0, 2):
        a0, a1, b0, b1 = h[i], l[i], h[i + 1], l[i + 1]
        c0 = mx(a0, b0)
        t1 = mn(a0, b0)
        t2 = mx(a1, b1)
        c3 = mn(a1, b1)
        c1 = mx(t1, t2)
        c2 = mn(t1, t2)
        quads.append((c0, c1, c2, c3))
    a, b = quads
    m = [mx(a[i], b[3 - i]) for i in range(4)]
    u0, u2 = mx(m[0], m[2]), mn(m[0], m[2])
    u1, u3 = mx(m[1], m[3]), mn(m[1], m[3])
    return mx(u0, u1), mn(u0, u1), mx(u2, u3), mn(u2, u3)


def _recruit_body(x_ref, p_ref, o_ref, ov_ref, *, top_k):
    x = x_ref[...]
    p = p_ref[0]
    ov_ref[...] = jnp.dot(x, p, preferred_element_type=jnp.float32)

    big = jnp.float32(3e38)
    rtot, n = ov_ref.shape
    n_sl = n // 128
    assert n_sl == 8, "fast path assumes 8 slices of 128 lanes"

    sub = 64

    def sub_body(s, worst):
        r0 = s * sub
        ovs = ov_ref[pl.ds(r0, sub), :]
        slices = [ovs[:, j * 128:(j + 1) * 128] for j in range(n_sl)]
        c1, c2, c3, c4 = _top4_of_8(slices)
        m = jnp.max(c1, axis=1, keepdims=True)
        for _ in range(top_k - 1):
            hit = c1 >= m
            c1 = jnp.where(hit, c2, c1)
            c2 = jnp.where(hit, c3, c2)
            c3 = jnp.where(hit, c4, c3)
            c4 = jnp.where(hit, -big, c4)
            m = jnp.max(c1, axis=1, keepdims=True)
        keep = ovs >= m
        o_ref[0, pl.ds(r0, sub), :] = jnp.where(keep, ovs, 0.0)
        nt = jnp.sum(jnp.where(keep, 1.0, 0.0), axis=1, keepdims=True)
        return jnp.maximum(worst, jnp.max(nt))

    worst = jax.lax.fori_loop(0, rtot // sub, sub_body, jnp.float32(0.0))

    anybad = worst != float(top_k)

    @pl.when(anybad)
    def _exact_recovery():
        ov = ov_ref[...]
        rows = ov.shape[0]
        vals = ov
        kk = jnp.full((rows, 1), float(top_k), dtype=jnp.float32)
        th = jnp.full((rows, 1), -big, dtype=jnp.float32)
        for _ in range(top_k):
            mm = jnp.max(vals, axis=1, keepdims=True)
            eq = vals >= mm
            c = jnp.sum(jnp.where(eq, 1.0, 0.0), axis=1, keepdims=True)
            take = (kk > 0.0) & (kk <= c)
            th = jnp.where(take, mm, th)
            kk = kk - c
            vals = jnp.where(eq, -big, vals)
        o_ref[0] = jnp.where(ov >= th, ov, 0.0)


def _recruit(x, P, row_block):
    B, D = x.shape
    C, _, N = P.shape
    top_k = int(np.ceil(SPARSITY * N))
    grid = (C, B // row_block)
    return pl.pallas_call(
        functools.partial(_recruit_body, top_k=top_k),
        grid=grid,
        in_specs=[
            pl.BlockSpec((row_block, D), lambda c, b: (b, 0)),
            pl.BlockSpec((1, D, N), lambda c, b: (c, 0, 0)),
        ],
        out_specs=pl.BlockSpec((1, row_block, N), lambda c, b: (c, b, 0)),
        out_shape=jax.ShapeDtypeStruct((C, B, N), jnp.float32),
        scratch_shapes=[pltpu.VMEM((row_block, N), jnp.float32)],
        compiler_params=pltpu.CompilerParams(
            dimension_semantics=("parallel", "parallel"),
        ),
    )(x, P)


def kernel(x, P):
    return _recruit(x, P, row_block=512)


# speedup vs baseline: 4.9268x; 4.9268x over previous
import functools

import jax
import jax.numpy as jnp
import numpy as np
from jax.experimental import pallas as pl
from jax.experimental.pallas import tpu as pltpu

SPARSITY = 0.02


def _top4_of_8(s):
    mx, mn = jnp.maximum, jnp.minimum
    h = [mx(s[2 * i], s[2 * i + 1]) for i in range(4)]
    l = [mn(s[2 * i], s[2 * i + 1]) for i in range(4)]
    quads = []
    for i in (0, 2):
        a0, a1, b0, b1 = h[i], l[i], h[i + 1], l[i + 1]
        c0 = mx(a0, b0)
        t1 = mn(a0, b0)
        t2 = mx(a1, b1)
        c3 = mn(a1, b1)
        c1 = mx(t1, t2)
        c2 = mn(t1, t2)
        quads.append((c0, c1, c2, c3))
    a, b = quads
    m = [mx(a[i], b[3 - i]) for i in range(4)]
    u0, u2 = mx(m[0], m[2]), mn(m[0], m[2])
    u1, u3 = mx(m[1], m[3]), mn(m[1], m[3])
    return mx(u0, u1), mn(u0, u1), mx(u2, u3), mn(u2, u3)


def _recruit_body(x_ref, p_ref, o_ref, *, top_k):
    x = x_ref[...]
    p = p_ref[0]
    ov = jnp.dot(x, p, preferred_element_type=jnp.float32)

    big = jnp.float32(3e38)
    n_sl = ov.shape[1] // 128
    assert n_sl == 8, "fast path assumes 8 slices of 128 lanes"
    slices = [ov[:, j * 128:(j + 1) * 128] for j in range(n_sl)]

    c1, c2, c3, c4 = _top4_of_8(slices)
    m = jnp.max(c1, axis=1, keepdims=True)
    for _ in range(top_k - 1):
        hit = c1 >= m
        c1 = jnp.where(hit, c2, c1)
        c2 = jnp.where(hit, c3, c2)
        c3 = jnp.where(hit, c4, c3)
        c4 = jnp.where(hit, -big, c4)
        m = jnp.max(c1, axis=1, keepdims=True)

    keep = ov >= m
    o_ref[0] = jnp.where(keep, ov, 0.0)
    nt = jnp.sum(jnp.where(keep, 1.0, 0.0), axis=1, keepdims=True)
    anybad = jnp.max(nt) != float(top_k)

    @pl.when(anybad)
    def _exact_recovery():
        rows = ov.shape[0]
        vals = ov
        kk = jnp.full((rows, 1), float(top_k), dtype=jnp.float32)
        th = jnp.full((rows, 1), -big, dtype=jnp.float32)
        for _ in range(top_k):
            mm = jnp.max(vals, axis=1, keepdims=True)
            eq = vals >= mm
            c = jnp.sum(jnp.where(eq, 1.0, 0.0), axis=1, keepdims=True)
            take = (kk > 0.0) & (kk <= c)
            th = jnp.where(take, mm, th)
            kk = kk - c
            vals = jnp.where(eq, -big, vals)
        o_ref[0] = jnp.where(ov >= th, ov, 0.0)


def _recruit(x, P, row_block):
    B, D = x.shape
    C, _, N = P.shape
    top_k = int(np.ceil(SPARSITY * N))
    grid = (C, B // row_block)
    return pl.pallas_call(
        functools.partial(_recruit_body, top_k=top_k),
        grid=grid,
        in_specs=[
            pl.BlockSpec((row_block, D), lambda c, b: (b, 0)),
            pl.BlockSpec((1, D, N), lambda c, b: (c, 0, 0)),
        ],
        out_specs=pl.BlockSpec((1, row_block, N), lambda c, b: (c, b, 0)),
        out_shape=jax.ShapeDtypeStruct((C, B, N), jnp.float32),
        compiler_params=pltpu.CompilerParams(
            dimension_semantics=("parallel", "parallel"),
        ),
    )(x, P)


def kernel(x, P):
    return _recruit(x, P, row_block=512)
